# baseline (speedup 1.0000x reference)
import jax
import jax.numpy as jnp
from jax import lax
from jax.experimental import pallas as pl
from jax.experimental.pallas import tpu as pltpu
from jax.experimental.pallas import tpu_sc as plsc

NUM_CORES = 2
NUM_SUBCORES = 16
NUM_WORKERS = NUM_CORES * NUM_SUBCORES

BATCH_SIZE = 16384
ROW_DIM = 64
VOCAB = 100000
DIMS_PER_WORKER = ROW_DIM // NUM_WORKERS
CHUNK = 4096
NUM_CHUNKS = BATCH_SIZE // CHUNK
LANES = 16


def _start_row_load(table_t, row_v, c, rsem):
    return [pltpu.async_copy(table_t.at[c], row_v, rsem)]


def _lookup_body(table_t, ids_hbm, out_t, ids_v, row_v, stage_a, stage_b,
                 isem, rsem, osem):
    wid = lax.axis_index("s") * NUM_CORES + lax.axis_index("c")
    c0 = wid * DIMS_PER_WORKER
    ids_cp = pltpu.async_copy(ids_hbm, ids_v, isem)
    row_cps = _start_row_load(table_t, row_v, c0, rsem)
    ids_cp.wait()
    stages = (stage_a, stage_b)
    pending = []
    for r in range(DIMS_PER_WORKER):
        c = c0 + r
        for cp in row_cps:
            cp.wait()
        for k in range(NUM_CHUNKS):
            stage = stages[k % 2]
            if len(pending) >= 2:
                pending.pop(0).wait()

            @plsc.parallel_loop(0, CHUNK, LANES, unroll=16)
            def gather_chunk(g, k=k, stage=stage):
                idx = ids_v[pl.ds(k * CHUNK + g, LANES)]
                stage[pl.ds(g, LANES)] = plsc.load_gather(row_v, [idx])

            pending.append(
                pltpu.async_copy(
                    stage, out_t.at[c, pl.ds(k * CHUNK, CHUNK)], osem
                )
            )
        if r + 1 < DIMS_PER_WORKER:
            row_cps = _start_row_load(table_t, row_v, c0 + r + 1, rsem)
    for w in pending:
        w.wait()


@jax.jit
def _lookup(table_t, ids):
    mesh = plsc.VectorSubcoreMesh(
        core_axis_name="c", subcore_axis_name="s",
        num_cores=NUM_CORES, num_subcores=NUM_SUBCORES,
    )
    fn = pl.kernel(
        _lookup_body,
        out_type=jax.ShapeDtypeStruct((ROW_DIM, BATCH_SIZE), jnp.float32),
        mesh=mesh,
        scratch_types=[
            pltpu.VMEM((BATCH_SIZE,), jnp.int32),
            pltpu.VMEM((VOCAB,), jnp.float32),
            pltpu.VMEM((CHUNK,), jnp.float32),
            pltpu.VMEM((CHUNK,), jnp.float32),
            pltpu.SemaphoreType.DMA,
            pltpu.SemaphoreType.DMA,
            pltpu.SemaphoreType.DMA,
        ],
        compiler_params=pltpu.CompilerParams(needs_layout_passes=False),
    )
    return fn(table_t, ids)


def kernel(speaker_ids, embedding_table):
    ids = speaker_ids.astype(jnp.int32)
    out_t = _lookup(embedding_table.T, ids)
    return out_t.T


# speedup vs baseline: 1.0043x; 1.0043x over previous
import jax
import jax.numpy as jnp
from jax import lax
from jax.experimental import pallas as pl
from jax.experimental.pallas import tpu as pltpu
from jax.experimental.pallas import tpu_sc as plsc

NUM_CORES = 2
NUM_SUBCORES = 16
NUM_WORKERS = NUM_CORES * NUM_SUBCORES

BATCH_SIZE = 16384
ROW_DIM = 64
VOCAB = 100000
DIMS_PER_WORKER = ROW_DIM // NUM_WORKERS
CHUNK = 4096
NUM_CHUNKS = BATCH_SIZE // CHUNK
LANES = 16


def _start_row_load(table_t, row_v, c, rsem):
    return [pltpu.async_copy(table_t.at[c], row_v, rsem)]


def _lookup_body(table_t, ids_hbm, out_t, ids_v, row_v, stage_a, stage_b,
                 isem, rsem, osem):
    wid = lax.axis_index("s") * NUM_CORES + lax.axis_index("c")
    c0 = wid * DIMS_PER_WORKER
    ids_cp = pltpu.async_copy(ids_hbm, ids_v, isem)
    row_cps = _start_row_load(table_t, row_v, c0, rsem)
    ids_cp.wait()
    stages = (stage_a, stage_b)
    pending = []
    for r in range(DIMS_PER_WORKER):
        c = c0 + r
        for cp in row_cps:
            cp.wait()
        for k in range(NUM_CHUNKS):
            stage = stages[k % 2]
            if len(pending) >= 2:
                pending.pop(0).wait()

            @plsc.parallel_loop(0, CHUNK, LANES, unroll=4)
            def gather_chunk(g, k=k, stage=stage):
                idx = ids_v[pl.ds(k * CHUNK + g, LANES)]
                stage[pl.ds(g, LANES)] = plsc.load_gather(row_v, [idx])

            pending.append(
                pltpu.async_copy(
                    stage, out_t.at[c, pl.ds(k * CHUNK, CHUNK)], osem
                )
            )
        if r + 1 < DIMS_PER_WORKER:
            row_cps = _start_row_load(table_t, row_v, c0 + r + 1, rsem)
    for w in pending:
        w.wait()


@jax.jit
def _lookup(table_t, ids):
    mesh = plsc.VectorSubcoreMesh(
        core_axis_name="c", subcore_axis_name="s",
        num_cores=NUM_CORES, num_subcores=NUM_SUBCORES,
    )
    fn = pl.kernel(
        _lookup_body,
        out_type=jax.ShapeDtypeStruct((ROW_DIM, BATCH_SIZE), jnp.float32),
        mesh=mesh,
        scratch_types=[
            pltpu.VMEM((BATCH_SIZE,), jnp.int32),
            pltpu.VMEM((VOCAB,), jnp.float32),
            pltpu.VMEM((CHUNK,), jnp.float32),
            pltpu.VMEM((CHUNK,), jnp.float32),
            pltpu.SemaphoreType.DMA,
            pltpu.SemaphoreType.DMA,
            pltpu.SemaphoreType.DMA,
        ],
        compiler_params=pltpu.CompilerParams(needs_layout_passes=False),
    )
    return fn(table_t, ids)


def kernel(speaker_ids, embedding_table):
    ids = speaker_ids.astype(jnp.int32)
    out_t = _lookup(embedding_table.T, ids)
    return out_t.T


# speedup vs baseline: 1.0054x; 1.0011x over previous
import jax
import jax.numpy as jnp
from jax import lax
from jax.experimental import pallas as pl
from jax.experimental.pallas import tpu as pltpu
from jax.experimental.pallas import tpu_sc as plsc

NUM_CORES = 2
NUM_SUBCORES = 16
NUM_WORKERS = NUM_CORES * NUM_SUBCORES

BATCH_SIZE = 16384
ROW_DIM = 64
VOCAB = 100000
DIMS_PER_WORKER = ROW_DIM // NUM_WORKERS
CHUNK = 4096
NUM_CHUNKS = BATCH_SIZE // CHUNK
LANES = 16


def _start_row_load(table_t, row_v, c, rsem):
    return [pltpu.async_copy(table_t.at[c], row_v, rsem)]


def _lookup_body(table_t, ids_hbm, out_t, ids_v, row_v, stage_a, stage_b,
                 isem, rsem, osem):
    wid = lax.axis_index("s") * NUM_CORES + lax.axis_index("c")
    c0 = wid * DIMS_PER_WORKER
    ids_cp = pltpu.async_copy(ids_hbm, ids_v, isem)
    row_cps = _start_row_load(table_t, row_v, c0, rsem)
    ids_cp.wait()
    stages = (stage_a, stage_b)
    pending = []
    for r in range(DIMS_PER_WORKER):
        c = c0 + r
        with jax.named_scope(f"rowwait{r}"):
            for cp in row_cps:
                cp.wait()
        with jax.named_scope(f"gather{r}"):
            for k in range(NUM_CHUNKS):
                stage = stages[k % 2]
                if len(pending) >= 2:
                    pending.pop(0).wait()

                @plsc.parallel_loop(0, CHUNK, LANES, unroll=4)
                def gather_chunk(g, k=k, stage=stage):
                    idx = ids_v[pl.ds(k * CHUNK + g, LANES)]
                    stage[pl.ds(g, LANES)] = plsc.load_gather(row_v, [idx])

                pending.append(
                    pltpu.async_copy(
                        stage, out_t.at[c, pl.ds(k * CHUNK, CHUNK)], osem
                    )
                )
        if r + 1 < DIMS_PER_WORKER:
            row_cps = _start_row_load(table_t, row_v, c0 + r + 1, rsem)
    with jax.named_scope("drain"):
        for w in pending:
            w.wait()


@jax.jit
def _lookup(table_t, ids):
    mesh = plsc.VectorSubcoreMesh(
        core_axis_name="c", subcore_axis_name="s",
        num_cores=NUM_CORES, num_subcores=NUM_SUBCORES,
    )
    fn = pl.kernel(
        _lookup_body,
        out_type=jax.ShapeDtypeStruct((ROW_DIM, BATCH_SIZE), jnp.float32),
        mesh=mesh,
        scratch_types=[
            pltpu.VMEM((BATCH_SIZE,), jnp.int32),
            pltpu.VMEM((VOCAB,), jnp.float32),
            pltpu.VMEM((CHUNK,), jnp.float32),
            pltpu.VMEM((CHUNK,), jnp.float32),
            pltpu.SemaphoreType.DMA,
            pltpu.SemaphoreType.DMA,
            pltpu.SemaphoreType.DMA,
        ],
        compiler_params=pltpu.CompilerParams(needs_layout_passes=False),
    )
    return fn(table_t, ids)


def kernel(speaker_ids, embedding_table):
    ids = speaker_ids.astype(jnp.int32)
    out_t = _lookup(embedding_table.T, ids)
    return out_t.T


# speedup vs baseline: 1.0062x; 1.0008x over previous
import jax
import jax.numpy as jnp
from jax import lax
from jax.experimental import pallas as pl
from jax.experimental.pallas import tpu as pltpu
from jax.experimental.pallas import tpu_sc as plsc

NUM_CORES = 2
NUM_SUBCORES = 16
NUM_WORKERS = NUM_CORES * NUM_SUBCORES

BATCH_SIZE = 16384
ROW_DIM = 64
VOCAB = 100000
DIMS_PER_WORKER = ROW_DIM // NUM_WORKERS
CHUNK = 4096
NUM_CHUNKS = BATCH_SIZE // CHUNK
LANES = 16


def _lookup_body(table_t, ids_hbm, out_t, ids_v, row_v, stage_a, stage_b,
                 isem, rsem, osem):
    wid = lax.axis_index("s") * NUM_CORES + lax.axis_index("c")
    c0 = wid * DIMS_PER_WORKER
    ids_cp = pltpu.async_copy(ids_hbm, ids_v, isem)
    row_cp = pltpu.async_copy(table_t.at[c0], row_v, rsem)
    ids_cp.wait()
    stages = (stage_a, stage_b)
    pending = []
    for r in range(DIMS_PER_WORKER):
        c = c0 + r
        row_cp.wait()
        for k in range(NUM_CHUNKS):
            stage = stages[k % 2]
            if len(pending) >= 2:
                pending.pop(0).wait()

            @plsc.parallel_loop(0, CHUNK, LANES, unroll=4)
            def gather_chunk(g, k=k, stage=stage):
                idx = ids_v[pl.ds(k * CHUNK + g, LANES)]
                stage[pl.ds(g, LANES)] = plsc.load_gather(row_v, [idx])

            pending.append(
                pltpu.async_copy(
                    stage, out_t.at[c, pl.ds(k * CHUNK, CHUNK)], osem
                )
            )
        if r + 1 < DIMS_PER_WORKER:
            row_cp = pltpu.async_copy(table_t.at[c0 + r + 1], row_v, rsem)
    for w in pending:
        w.wait()


@jax.jit
def _lookup(table_t, ids):
    mesh = plsc.VectorSubcoreMesh(
        core_axis_name="c", subcore_axis_name="s",
        num_cores=NUM_CORES, num_subcores=NUM_SUBCORES,
    )
    fn = pl.kernel(
        _lookup_body,
        out_type=jax.ShapeDtypeStruct((ROW_DIM, BATCH_SIZE), jnp.float32),
        mesh=mesh,
        scratch_types=[
            pltpu.VMEM((BATCH_SIZE,), jnp.int32),
            pltpu.VMEM((VOCAB,), jnp.float32),
            pltpu.VMEM((CHUNK,), jnp.float32),
            pltpu.VMEM((CHUNK,), jnp.float32),
            pltpu.SemaphoreType.DMA,
            pltpu.SemaphoreType.DMA,
            pltpu.SemaphoreType.DMA,
        ],
        compiler_params=pltpu.CompilerParams(needs_layout_passes=False),
    )
    return fn(table_t, ids)


def kernel(speaker_ids, embedding_table):
    ids = speaker_ids.astype(jnp.int32)
    out_t = _lookup(embedding_table.T, ids)
    return out_t.T


# speedup vs baseline: 1.0096x; 1.0034x over previous
import jax
import jax.numpy as jnp
from jax import lax
from jax.experimental import pallas as pl
from jax.experimental.pallas import tpu as pltpu
from jax.experimental.pallas import tpu_sc as plsc

NUM_CORES = 2
NUM_SUBCORES = 16
NUM_WORKERS = NUM_CORES * NUM_SUBCORES

BATCH_SIZE = 16384
ROW_DIM = 64
VOCAB = 100000
DIMS_PER_WORKER = ROW_DIM // NUM_WORKERS
CHUNK = 4096
NUM_CHUNKS = BATCH_SIZE // CHUNK
LANES = 16


def _lookup_body(table_t, ids_hbm, out_t, ids_v, row_v, stage_a, stage_b,
                 isem, rsem, osem):
    wid = lax.axis_index("s") * NUM_CORES + lax.axis_index("c")
    c0 = wid * DIMS_PER_WORKER
    row_cp = pltpu.async_copy(table_t.at[c0], row_v, rsem)
    ids_cp = pltpu.async_copy(ids_hbm, ids_v, isem)
    ids_cp.wait()
    stages = (stage_a, stage_b)
    pending = []
    for r in range(DIMS_PER_WORKER):
        c = c0 + r
        row_cp.wait()
        for k in range(NUM_CHUNKS):
            stage = stages[k % 2]
            if len(pending) >= 2:
                pending.pop(0).wait()

            @plsc.parallel_loop(0, CHUNK, LANES, unroll=8)
            def gather_chunk(g, k=k, stage=stage):
                idx = ids_v[pl.ds(k * CHUNK + g, LANES)]
                stage[pl.ds(g, LANES)] = plsc.load_gather(row_v, [idx])

            pending.append(
                pltpu.async_copy(
                    stage, out_t.at[c, pl.ds(k * CHUNK, CHUNK)], osem
                )
            )
        if r + 1 < DIMS_PER_WORKER:
            row_cp = pltpu.async_copy(table_t.at[c0 + r + 1], row_v, rsem)
    for w in pending:
        w.wait()


@jax.jit
def _lookup(table_t, ids):
    mesh = plsc.VectorSubcoreMesh(
        core_axis_name="c", subcore_axis_name="s",
        num_cores=NUM_CORES, num_subcores=NUM_SUBCORES,
    )
    fn = pl.kernel(
        _lookup_body,
        out_type=jax.ShapeDtypeStruct((ROW_DIM, BATCH_SIZE), jnp.float32),
        mesh=mesh,
        scratch_types=[
            pltpu.VMEM((BATCH_SIZE,), jnp.int32),
            pltpu.VMEM((VOCAB,), jnp.float32),
            pltpu.VMEM((CHUNK,), jnp.float32),
            pltpu.VMEM((CHUNK,), jnp.float32),
            pltpu.SemaphoreType.DMA,
            pltpu.SemaphoreType.DMA,
            pltpu.SemaphoreType.DMA,
        ],
        compiler_params=pltpu.CompilerParams(needs_layout_passes=False),
    )
    return fn(table_t, ids)


def kernel(speaker_ids, embedding_table):
    ids = speaker_ids.astype(jnp.int32)
    out_t = _lookup(embedding_table.T, ids)
    return out_t.T


# speedup vs baseline: 1.0105x; 1.0008x over previous
import jax
import jax.numpy as jnp
from jax import lax
from jax.experimental import pallas as pl
from jax.experimental.pallas import tpu as pltpu
from jax.experimental.pallas import tpu_sc as plsc

NUM_CORES = 2
NUM_SUBCORES = 16
NUM_WORKERS = NUM_CORES * NUM_SUBCORES

BATCH_SIZE = 16384
ROW_DIM = 64
VOCAB = 100000
DIMS_PER_WORKER = ROW_DIM // NUM_WORKERS
CHUNK = 4096
NUM_CHUNKS = BATCH_SIZE // CHUNK
LANES = 16


def _lookup_body(table_t, ids_hbm, out_t, ids_v, row_v, stage_a, stage_b,
                 isem, rsem, osem):
    wid = lax.axis_index("s") * NUM_CORES + lax.axis_index("c")
    c0 = wid * DIMS_PER_WORKER
    row_cp = pltpu.async_copy(table_t.at[c0], row_v, rsem)
    ids_cp = pltpu.async_copy(ids_hbm, ids_v, isem)
    ids_cp.wait()
    stages = (stage_a, stage_b)
    pending = []
    for r in range(DIMS_PER_WORKER):
        c = c0 + r
        row_cp.wait()
        for k in range(NUM_CHUNKS):
            stage = stages[k % 2]
            if len(pending) >= 2:
                pending.pop(0).wait()

            @plsc.parallel_loop(0, CHUNK, LANES, unroll=8)
            def gather_chunk(g, k=k, stage=stage):
                idx = ids_v[pl.ds(k * CHUNK + g, LANES)]
                stage[pl.ds(g, LANES)] = plsc.load_gather(row_v, [idx])

            pending.append(
                pltpu.async_copy(
                    stage, out_t.at[c, pl.ds(k * CHUNK, CHUNK)], osem
                )
            )
        if r + 1 < DIMS_PER_WORKER:
            row_cp = pltpu.async_copy(table_t.at[c0 + r + 1], row_v, rsem)
    for w in pending:
        w.wait()


@jax.jit
def _lookup(table_t, ids):
    mesh = plsc.VectorSubcoreMesh(
        core_axis_name="c", subcore_axis_name="s",
        num_cores=NUM_CORES, num_subcores=NUM_SUBCORES,
    )
    fn = pl.kernel(
        _lookup_body,
        out_type=jax.ShapeDtypeStruct((ROW_DIM, BATCH_SIZE), jnp.float32),
        mesh=mesh,
        scratch_types=[
            pltpu.VMEM((BATCH_SIZE,), jnp.int32),
            pltpu.VMEM((VOCAB,), jnp.float32),
            pltpu.VMEM((CHUNK,), jnp.float32),
            pltpu.VMEM((CHUNK,), jnp.float32),
            pltpu.SemaphoreType.DMA,
            pltpu.SemaphoreType.DMA,
            pltpu.SemaphoreType.DMA,
        ],
        compiler_params=pltpu.CompilerParams(
            needs_layout_passes=False,
            disable_bounds_checks=True,
            disable_semaphore_checks=True,
            skip_device_barrier=True,
        ),
    )
    return fn(table_t, ids)


def kernel(speaker_ids, embedding_table):
    ids = speaker_ids.astype(jnp.int32)
    out_t = _lookup(embedding_table.T, ids)
    return out_t.T
